# baseline (speedup 1.0000x reference)
import functools

import jax
import jax.numpy as jnp
from jax.experimental import pallas as pl
from jax.experimental.pallas import tpu as pltpu

N_ROWS = 16384
HIDDEN = 1024
TABLE_ROWS = 1000

B = 1024
NSTEPS = N_ROWS // B
NBUF = 4
LA = NBUF - 1
HALF = B // 2
SUB = 16


def _body(idx_smem, feat_any, ivec_any, table_any, out_any,
          tbl_v, fin, ivin, fout, fsem, isem, osem, tsem):
    s = pl.program_id(0)

    def issue_in(step):
        b = step % NBUF
        for h in range(2):
            pltpu.async_copy(
                feat_any.at[pl.ds(step * B + h * HALF, HALF)],
                fin.at[b, pl.ds(h * HALF, HALF)], fsem.at[b, h])
        pltpu.async_copy(ivec_any.at[pl.ds(step * B, B)],
                         ivin.at[b], isem.at[b])

    @pl.when(s == 0)
    def _():
        tcp = pltpu.async_copy(table_any, tbl_v, tsem)
        for st in range(LA):
            issue_in(st)
        tcp.wait()

    @pl.when(s + LA < NSTEPS)
    def _():
        issue_in(s + LA)

    b = s % NBUF

    @pl.when(s >= NBUF)
    def _():
        for h in range(2):
            pltpu.make_async_copy(
                fout.at[b, pl.ds(h * HALF, HALF)],
                out_any.at[pl.ds(h * HALF, HALF)], osem.at[b, h]).wait()

    for h in range(2):
        pltpu.make_async_copy(
            feat_any.at[pl.ds(h * HALF, HALF)],
            fin.at[b, pl.ds(h * HALF, HALF)], fsem.at[b, h]).wait()
    pltpu.make_async_copy(ivec_any.at[pl.ds(0, B)], ivin.at[b],
                          isem.at[b]).wait()

    for k in range(B // SUB):
        base = s * B + k * SUB
        r0 = idx_smem[base]
        r1 = idx_smem[base + SUB - 1]
        a = tbl_v[pl.ds(r0, 1), :]
        c = tbl_v[pl.ds(r1, 1), :]
        idx_v = ivin[b, pl.ds(k * SUB, SUB), :]
        mask = idx_v == r0
        sl = pl.ds(k * SUB, SUB)
        fout[b, sl, :] = fin[b, sl, :] + jnp.where(mask, a, c)

    for h in range(2):
        pltpu.async_copy(
            fout.at[b, pl.ds(h * HALF, HALF)],
            out_any.at[pl.ds(s * B + h * HALF, HALF)], osem.at[b, h])

    @pl.when(s == NSTEPS - 1)
    def _():
        for d in range(NBUF):
            bb = (s - d) % NBUF
            for h in range(2):
                pltpu.make_async_copy(
                    fout.at[bb, pl.ds(h * HALF, HALF)],
                    out_any.at[pl.ds(h * HALF, HALF)], osem.at[bb, h]).wait()


@jax.jit
def kernel(features, temporal_embedding):
    n = features.shape[0]
    temporal_pos = jnp.linspace(0.0, 1.0, n, dtype=features.dtype)
    idx = jnp.clip(temporal_pos * TABLE_ROWS, 0, TABLE_ROWS - 1).astype(jnp.int32)
    idx_vec = idx.reshape(n, 1)

    grid_spec = pltpu.PrefetchScalarGridSpec(
        num_scalar_prefetch=1,
        grid=(NSTEPS,),
        in_specs=[
            pl.BlockSpec(memory_space=pl.ANY),
            pl.BlockSpec(memory_space=pl.ANY),
            pl.BlockSpec(memory_space=pl.ANY),
        ],
        out_specs=pl.BlockSpec(memory_space=pl.ANY),
        scratch_shapes=[
            pltpu.VMEM((TABLE_ROWS, HIDDEN), jnp.float32),
            pltpu.VMEM((NBUF, B, HIDDEN), jnp.float32),
            pltpu.VMEM((NBUF, B, 1), jnp.int32),
            pltpu.VMEM((NBUF, B, HIDDEN), jnp.float32),
            pltpu.SemaphoreType.DMA((NBUF, 2)),
            pltpu.SemaphoreType.DMA((NBUF,)),
            pltpu.SemaphoreType.DMA((NBUF, 2)),
            pltpu.SemaphoreType.DMA,
        ],
    )
    return pl.pallas_call(
        _body,
        grid_spec=grid_spec,
        out_shape=jax.ShapeDtypeStruct((n, HIDDEN), features.dtype),
        compiler_params=pltpu.CompilerParams(
            dimension_semantics=("arbitrary",)),
    )(idx, features, idx_vec, temporal_embedding)


# speedup vs baseline: 1.0158x; 1.0158x over previous
import functools

import jax
import jax.numpy as jnp
from jax.experimental import pallas as pl
from jax.experimental.pallas import tpu as pltpu

N_ROWS = 16384
HIDDEN = 1024
TABLE_ROWS = 1000

B = 2048
NSTEPS = N_ROWS // B
NBUF = 3
LA = NBUF - 1
HALF = B // 2
SUB = 16


def _body(idx_smem, feat_any, ivec_any, table_any, out_any,
          tbl_v, fin, ivin, fout, fsem, isem, osem, tsem):
    s = pl.program_id(0)

    def issue_in(step):
        b = step % NBUF
        for h in range(2):
            pltpu.async_copy(
                feat_any.at[pl.ds(step * B + h * HALF, HALF)],
                fin.at[b, pl.ds(h * HALF, HALF)], fsem.at[b, h])
        pltpu.async_copy(ivec_any.at[pl.ds(step * B, B)],
                         ivin.at[b], isem.at[b])

    @pl.when(s == 0)
    def _():
        tcp = pltpu.async_copy(table_any, tbl_v, tsem)
        for st in range(LA):
            issue_in(st)
        tcp.wait()

    @pl.when(s + LA < NSTEPS)
    def _():
        issue_in(s + LA)

    b = s % NBUF

    @pl.when(s >= NBUF)
    def _():
        for h in range(2):
            pltpu.make_async_copy(
                fout.at[b, pl.ds(h * HALF, HALF)],
                out_any.at[pl.ds(h * HALF, HALF)], osem.at[b, h]).wait()

    for h in range(2):
        pltpu.make_async_copy(
            feat_any.at[pl.ds(h * HALF, HALF)],
            fin.at[b, pl.ds(h * HALF, HALF)], fsem.at[b, h]).wait()
    pltpu.make_async_copy(ivec_any.at[pl.ds(0, B)], ivin.at[b],
                          isem.at[b]).wait()

    for k in range(B // SUB):
        base = s * B + k * SUB
        r0 = idx_smem[base]
        r1 = idx_smem[base + SUB - 1]
        a = tbl_v[pl.ds(r0, 1), :]
        c = tbl_v[pl.ds(r1, 1), :]
        idx_v = ivin[b, pl.ds(k * SUB, SUB), :]
        mask = idx_v == r0
        sl = pl.ds(k * SUB, SUB)
        fout[b, sl, :] = fin[b, sl, :] + jnp.where(mask, a, c)

    for h in range(2):
        pltpu.async_copy(
            fout.at[b, pl.ds(h * HALF, HALF)],
            out_any.at[pl.ds(s * B + h * HALF, HALF)], osem.at[b, h])

    @pl.when(s == NSTEPS - 1)
    def _():
        for d in range(NBUF):
            bb = (s - d) % NBUF
            for h in range(2):
                pltpu.make_async_copy(
                    fout.at[bb, pl.ds(h * HALF, HALF)],
                    out_any.at[pl.ds(h * HALF, HALF)], osem.at[bb, h]).wait()


@jax.jit
def kernel(features, temporal_embedding):
    n = features.shape[0]
    temporal_pos = jnp.linspace(0.0, 1.0, n, dtype=features.dtype)
    idx = jnp.clip(temporal_pos * TABLE_ROWS, 0, TABLE_ROWS - 1).astype(jnp.int32)
    idx_vec = idx.reshape(n, 1)

    grid_spec = pltpu.PrefetchScalarGridSpec(
        num_scalar_prefetch=1,
        grid=(NSTEPS,),
        in_specs=[
            pl.BlockSpec(memory_space=pl.ANY),
            pl.BlockSpec(memory_space=pl.ANY),
            pl.BlockSpec(memory_space=pl.ANY),
        ],
        out_specs=pl.BlockSpec(memory_space=pl.ANY),
        scratch_shapes=[
            pltpu.VMEM((TABLE_ROWS, HIDDEN), jnp.float32),
            pltpu.VMEM((NBUF, B, HIDDEN), jnp.float32),
            pltpu.VMEM((NBUF, B, 1), jnp.int32),
            pltpu.VMEM((NBUF, B, HIDDEN), jnp.float32),
            pltpu.SemaphoreType.DMA((NBUF, 2)),
            pltpu.SemaphoreType.DMA((NBUF,)),
            pltpu.SemaphoreType.DMA((NBUF, 2)),
            pltpu.SemaphoreType.DMA,
        ],
    )
    return pl.pallas_call(
        _body,
        grid_spec=grid_spec,
        out_shape=jax.ShapeDtypeStruct((n, HIDDEN), features.dtype),
        compiler_params=pltpu.CompilerParams(
            dimension_semantics=("arbitrary",)),
    )(idx, features, idx_vec, temporal_embedding)
